# baseline (speedup 1.0000x reference)
import functools
import jax
import jax.numpy as jnp
from jax import lax
from jax.experimental import pallas as pl
from jax.experimental.pallas import tpu as pltpu
from jax.experimental.pallas import tpu_sc as plsc

D = 64
SCALE = 8.0
NC = 2
NS = 16
LANES = 16


def kernel(indices, table):
    B = indices.shape[0] * indices.shape[1]
    NW = NC * NS
    per_w = B // NW
    C = 512
    n_chunks = per_w // C

    idx_flat = indices.reshape(B).astype(jnp.int32)

    mesh = plsc.VectorSubcoreMesh(core_axis_name="c", subcore_axis_name="s")

    @functools.partial(
        pl.kernel,
        out_type=jax.ShapeDtypeStruct((B, D), jnp.float32),
        mesh=mesh,
        scratch_types=[
            pltpu.VMEM((C,), jnp.int32),
            pltpu.VMEM((C, D), jnp.float32),
            pltpu.SemaphoreType.DMA,
        ],
        compiler_params=pltpu.CompilerParams(use_tc_tiling_on_sc=False),
    )
    def emb(idx_hbm, table_hbm, out_hbm, idx_v, rows_v, sem):
        wid = lax.axis_index("s") * NC + lax.axis_index("c")
        base = wid * per_w

        def chunk_body(g, carry):
            off = base + g * C
            pltpu.sync_copy(idx_hbm.at[pl.ds(off, C)], idx_v)
            pltpu.async_copy(table_hbm.at[idx_v], rows_v, sem).wait()

            def scale_row(i, carry2):
                for j in range(D // LANES):
                    sl = pl.ds(j * LANES, LANES)
                    rows_v[i, sl] = rows_v[i, sl] * SCALE
                return carry2

            lax.fori_loop(0, C, scale_row, 0)
            pltpu.sync_copy(rows_v, out_hbm.at[pl.ds(off, C)])
            return carry

        lax.fori_loop(0, n_chunks, chunk_body, 0)

    out = emb(idx_flat, table)
    return out.reshape(indices.shape[0], indices.shape[1], D)


# speedup vs baseline: 1.1377x; 1.1377x over previous
import functools
import jax
import jax.numpy as jnp
from jax import lax
from jax.experimental import pallas as pl
from jax.experimental.pallas import tpu as pltpu
from jax.experimental.pallas import tpu_sc as plsc

D = 64
SCALE = 8.0
NC = 2
NS = 16
LANES = 16

NBUF = 4
LOOKAHEAD = 2


def kernel(indices, table):
    B = indices.shape[0] * indices.shape[1]
    NW = NC * NS
    per_w = B // NW
    C = 256
    n_chunks = per_w // C
    n_groups = n_chunks // NBUF

    idx_flat = indices.reshape(B).astype(jnp.int32)

    mesh = plsc.VectorSubcoreMesh(core_axis_name="c", subcore_axis_name="s")

    @functools.partial(
        pl.kernel,
        out_type=jax.ShapeDtypeStruct((B, D), jnp.float32),
        mesh=mesh,
        scratch_types=(
            [pltpu.VMEM((per_w,), jnp.int32)]
            + [pltpu.VMEM((C, D), jnp.float32) for _ in range(NBUF)]
            + [pltpu.SemaphoreType.DMA for _ in range(NBUF)]
            + [pltpu.SemaphoreType.DMA for _ in range(NBUF)]
        ),
        compiler_params=pltpu.CompilerParams(use_tc_tiling_on_sc=False),
    )
    def emb(idx_hbm, table_hbm, out_hbm, idx_v, *bufs_and_sems):
        bufs = bufs_and_sems[:NBUF]
        gsem = bufs_and_sems[NBUF:2 * NBUF]
        ssem = bufs_and_sems[2 * NBUF:3 * NBUF]

        wid = lax.axis_index("s") * NC + lax.axis_index("c")
        base = wid * per_w

        pltpu.sync_copy(idx_hbm.at[pl.ds(base, per_w)], idx_v)

        def issue_gather(g, b):
            pltpu.async_copy(
                table_hbm.at[idx_v.at[pl.ds(g * C, C)]], bufs[b], gsem[b])

        def wait_gather(g, b):
            pltpu.make_async_copy(
                table_hbm.at[idx_v.at[pl.ds(g * C, C)]], bufs[b],
                gsem[b]).wait()

        def issue_store(g, b):
            pltpu.async_copy(
                bufs[b], out_hbm.at[pl.ds(base + g * C, C)], ssem[b])

        def wait_store(g, b):
            pltpu.make_async_copy(
                bufs[b], out_hbm.at[pl.ds(base + g * C, C)], ssem[b]).wait()

        def scale(b):
            buf = bufs[b]

            @plsc.parallel_loop(0, C, 1, unroll=4)
            def _(i):
                for j in range(D // LANES):
                    sl = pl.ds(j * LANES, LANES)
                    buf[i, sl] = buf[i, sl] * SCALE

        for g in range(LOOKAHEAD):
            issue_gather(g, g)

        def group_body(gi, carry):
            for p in range(NBUF):
                g = gi * NBUF + p
                q = (p + LOOKAHEAD) % NBUF
                wait_gather(g, p)
                scale(p)
                issue_store(g, p)
                if p < NBUF - LOOKAHEAD:
                    @pl.when(gi >= 1)
                    def _():
                        wait_store(g - LOOKAHEAD, q)
                    issue_gather(g + LOOKAHEAD, q)
                else:
                    @pl.when(gi < n_groups - 1)
                    def _():
                        wait_store(g - LOOKAHEAD, q)
                        issue_gather(g + LOOKAHEAD, q)
            return carry

        lax.fori_loop(0, n_groups, group_body, 0)

        for p in range(NBUF):
            wait_store(n_chunks - NBUF + p, p)

    out = emb(idx_flat, table)
    return out.reshape(indices.shape[0], indices.shape[1], D)
